# baseline (speedup 1.0000x reference)
import functools

import jax
import jax.numpy as jnp
from jax.experimental import pallas as pl
from jax.experimental.pallas import tpu as pltpu

B, S, D, DFF, E, K = 1, 2048, 768, 3072, 8, 2
N = B * S
TILE_N = 256


def _router_shared_body(x_ref, Wr_ref, br_ref, W1s_ref, b1s_ref, W2s_ref,
                        b2s_ref, shared_ref, w1_ref, w2_ref, i1_ref, i2_ref):
    x = x_ref[...]
    logits = jax.lax.dot_general(
        x, Wr_ref[...], (((1,), (1,)), ((), ())),
        preferred_element_type=jnp.float32) + br_ref[...]
    idx = jax.lax.broadcasted_iota(jnp.int32, logits.shape, 1)
    m1 = jnp.max(logits, axis=-1, keepdims=True)
    i1 = jnp.min(jnp.where(logits == m1, idx, E), axis=-1, keepdims=True)
    masked = jnp.where(idx == i1, -jnp.inf, logits)
    m2 = jnp.max(masked, axis=-1, keepdims=True)
    i2 = jnp.min(jnp.where(masked == m2, idx, E), axis=-1, keepdims=True)
    b = jnp.exp(m2 - m1)
    w1 = 1.0 / (1.0 + b)
    w1_ref[...] = w1
    w2_ref[...] = 1.0 - w1
    i1_ref[...] = i1
    i2_ref[...] = i2
    h = jax.nn.gelu(jax.lax.dot_general(
        x, W1s_ref[...], (((1,), (1,)), ((), ())),
        preferred_element_type=jnp.float32) + b1s_ref[...])
    shared_ref[...] = jax.lax.dot_general(
        h, W2s_ref[...], (((1,), (1,)), ((), ())),
        preferred_element_type=jnp.float32) + b2s_ref[...]


def _experts_body(x_ref, w1_ref, w2_ref, i1_ref, i2_ref, shared_ref,
                  W1e_ref, b1e_ref, W2e_ref, b2e_ref, out_ref):
    e = pl.program_id(1)

    @pl.when(e == 0)
    def _():
        out_ref[...] = shared_ref[...]

    we = (jnp.where(i1_ref[...] == e, w1_ref[...], 0.0)
          + jnp.where(i2_ref[...] == e, w2_ref[...], 0.0))
    x = x_ref[...]
    h = jax.nn.gelu(jax.lax.dot_general(
        x, W1e_ref[0], (((1,), (0,)), ((), ())),
        preferred_element_type=jnp.float32) + b1e_ref[0])
    y = jax.lax.dot_general(
        h, W2e_ref[0], (((1,), (0,)), ((), ())),
        preferred_element_type=jnp.float32) + b2e_ref[0]
    out_ref[...] += we * y


@jax.jit
def kernel(x, Wr, br, W1s, b1s, W2s, b2s, W1e, b1e, W2e, b2e):
    flat = x.reshape(N, D)
    nt = N // TILE_N

    shared, w1, w2, i1, i2 = pl.pallas_call(
        _router_shared_body,
        grid=(nt,),
        in_specs=[
            pl.BlockSpec((TILE_N, D), lambda t: (t, 0)),
            pl.BlockSpec((E, D), lambda t: (0, 0)),
            pl.BlockSpec((1, E), lambda t: (0, 0)),
            pl.BlockSpec((DFF, D), lambda t: (0, 0)),
            pl.BlockSpec((1, DFF), lambda t: (0, 0)),
            pl.BlockSpec((D, DFF), lambda t: (0, 0)),
            pl.BlockSpec((1, D), lambda t: (0, 0)),
        ],
        out_specs=[
            pl.BlockSpec((TILE_N, D), lambda t: (t, 0)),
            pl.BlockSpec((TILE_N, 1), lambda t: (t, 0)),
            pl.BlockSpec((TILE_N, 1), lambda t: (t, 0)),
            pl.BlockSpec((TILE_N, 1), lambda t: (t, 0)),
            pl.BlockSpec((TILE_N, 1), lambda t: (t, 0)),
        ],
        out_shape=[
            jax.ShapeDtypeStruct((N, D), jnp.float32),
            jax.ShapeDtypeStruct((N, 1), jnp.float32),
            jax.ShapeDtypeStruct((N, 1), jnp.float32),
            jax.ShapeDtypeStruct((N, 1), jnp.int32),
            jax.ShapeDtypeStruct((N, 1), jnp.int32),
        ],
        compiler_params=pltpu.CompilerParams(
            dimension_semantics=("parallel",),
        ),
    )(flat, Wr, br.reshape(1, E), W1s, b1s.reshape(1, DFF),
      W2s, b2s.reshape(1, D))

    out = pl.pallas_call(
        _experts_body,
        grid=(nt, E),
        in_specs=[
            pl.BlockSpec((TILE_N, D), lambda t, e: (t, 0)),
            pl.BlockSpec((TILE_N, 1), lambda t, e: (t, 0)),
            pl.BlockSpec((TILE_N, 1), lambda t, e: (t, 0)),
            pl.BlockSpec((TILE_N, 1), lambda t, e: (t, 0)),
            pl.BlockSpec((TILE_N, 1), lambda t, e: (t, 0)),
            pl.BlockSpec((TILE_N, D), lambda t, e: (t, 0)),
            pl.BlockSpec((1, D, DFF), lambda t, e: (e, 0, 0)),
            pl.BlockSpec((1, 1, DFF), lambda t, e: (e, 0, 0)),
            pl.BlockSpec((1, DFF, D), lambda t, e: (e, 0, 0)),
            pl.BlockSpec((1, 1, D), lambda t, e: (e, 0, 0)),
        ],
        out_specs=pl.BlockSpec((TILE_N, D), lambda t, e: (t, 0)),
        out_shape=jax.ShapeDtypeStruct((N, D), jnp.float32),
        compiler_params=pltpu.CompilerParams(
            dimension_semantics=("parallel", "arbitrary"),
        ),
    )(flat, w1, w2, i1, i2, shared, W1e, b1e.reshape(E, 1, DFF), W2e,
      b2e.reshape(E, 1, D))
    return out.reshape(B, S, D)


# speedup vs baseline: 1.9559x; 1.9559x over previous
import functools

import jax
import jax.numpy as jnp
from jax import lax
from jax.experimental import pallas as pl
from jax.experimental.pallas import tpu as pltpu
from jax.experimental.pallas import tpu_sc as plsc

B, S, D, DFF, E, K = 1, 2048, 768, 3072, 8, 2
N = B * S
P = K * N
TILE = 256
MAX_TILES = 24
P_PAD = MAX_TILES * TILE
NC, NS, L = 2, 16, 16
NW = NC * NS
TPW = N // NW
PPW = P // NW
_sc_mesh = plsc.VectorSubcoreMesh(
    core_axis_name="c", subcore_axis_name="s", num_cores=NC, num_subcores=NS)


def _router_shared_body(x_ref, Wr_ref, br_ref, W1s_ref, b1s_ref, W2s_ref,
                        b2s_ref, shared_ref, w1_ref, w2_ref, i1_ref, i2_ref):
    x = x_ref[...]
    logits = lax.dot_general(
        x, Wr_ref[...], (((1,), (1,)), ((), ())),
        preferred_element_type=jnp.float32) + br_ref[...]
    idx = lax.broadcasted_iota(jnp.int32, logits.shape, 1)
    m1 = jnp.max(logits, axis=-1, keepdims=True)
    i1 = jnp.min(jnp.where(logits == m1, idx, E), axis=-1, keepdims=True)
    masked = jnp.where(idx == i1, -jnp.inf, logits)
    m2 = jnp.max(masked, axis=-1, keepdims=True)
    i2 = jnp.min(jnp.where(masked == m2, idx, E), axis=-1, keepdims=True)
    b = jnp.exp(m2 - m1)
    w1 = 1.0 / (1.0 + b)
    w1_ref[...] = w1
    w2_ref[...] = 1.0 - w1
    i1_ref[...] = i1
    i2_ref[...] = i2
    h = jax.nn.gelu(lax.dot_general(
        x, W1s_ref[...], (((1,), (1,)), ((), ())),
        preferred_element_type=jnp.float32) + b1s_ref[...])
    shared_ref[...] = lax.dot_general(
        h, W2s_ref[...], (((1,), (1,)), ((), ())),
        preferred_element_type=jnp.float32) + b2s_ref[...]


def _lane_splat(tmp_ref, vec, lane):
    tmp_ref[...] = vec
    return plsc.load_gather(tmp_ref, [jnp.full((L,), lane, jnp.int32)])


def _build_body(ids_hbm, slot_hbm, te_hbm, ids_v, slot_v, te_v):
    cid = lax.axis_index("c")
    w = lax.axis_index("s") * NC + cid
    lane = lax.iota(jnp.int32, L)
    pltpu.sync_copy(ids_hbm, ids_v)

    def count_upto(hi):
        def body(j, tv):
            v = ids_v[pl.ds(j * L, L)]
            for e in range(E):
                s = jnp.sum(jnp.where(v == e, 1, 0).astype(jnp.int32))
                tv = tv + jnp.where(lane == e, s, 0)
            return tv
        return lax.fori_loop(0, hi, body, jnp.zeros((L,), jnp.int32))

    tot_v = count_upto(P // L)
    pref_v = count_upto(w * (PPW // L))

    base = []
    off = []
    acc = jnp.int32(0)
    for e in range(E):
        cnt = jnp.sum(jnp.where(lane == e, tot_v, 0))
        prefw = jnp.sum(jnp.where(lane == e, pref_v, 0))
        off.append(acc)
        base.append(acc + prefw)
        acc = acc + ((cnt + (TILE - 1)) // TILE) * TILE

    for i in range(PPW // L):
        v = ids_v[pl.ds(w * PPW + i * L, L)]
        slots = jnp.zeros((L,), jnp.int32)
        for e in range(E):
            mask = v == e
            mi = jnp.where(mask, 1, 0).astype(jnp.int32)
            ranks = plsc.cumsum(mi)
            slots = jnp.where(mask, base[e] + ranks - 1, slots)
            base[e] = base[e] + jnp.sum(mi)
        slot_v[pl.ds(i * L, L)] = slots
    pltpu.sync_copy(slot_v, slot_hbm.at[pl.ds(w * PPW, PPW)])

    @pl.when(w == 0)
    def _tiles():
        t0 = lane * TILE
        t1 = (lane + NS) * TILE
        te0 = jnp.full((L,), -1, jnp.int32)
        te1 = jnp.full((L,), -1, jnp.int32)
        for e in range(E):
            te0 = te0 + jnp.where(t0 >= off[e], 1, 0)
            te1 = te1 + jnp.where(t1 >= off[e], 1, 0)
        te_v[pl.ds(0, L)] = te0
        te_v[pl.ds(L, L)] = te1
        pltpu.sync_copy(te_v, te_hbm)


def _dispatch_body(x_hbm, slot_hbm, xs_hbm, idx0_v, idx1_v, xbuf, sem):
    wid = lax.axis_index("s") * NC + lax.axis_index("c")
    base = wid * TPW
    pltpu.sync_copy(slot_hbm.at[pl.ds(base, TPW)], idx0_v)
    pltpu.sync_copy(slot_hbm.at[pl.ds(N + base, TPW)], idx1_v)
    pltpu.sync_copy(x_hbm.at[pl.ds(base, TPW)], xbuf)
    pltpu.async_copy(xbuf, xs_hbm.at[idx0_v], sem).wait()
    pltpu.async_copy(xbuf, xs_hbm.at[idx1_v], sem).wait()


def _experts_body(te_ref, xs_ref, W1e_ref, b1e_ref, W2e_ref, b2e_ref, y_ref):
    h = jax.nn.gelu(lax.dot_general(
        xs_ref[...], W1e_ref[0], (((1,), (0,)), ((), ())),
        preferred_element_type=jnp.float32) + b1e_ref[0])
    y_ref[...] = lax.dot_general(
        h, W2e_ref[0], (((1,), (0,)), ((), ())),
        preferred_element_type=jnp.float32) + b2e_ref[0]


def _combine_body(shared_hbm, y_hbm, tw_hbm, slot_hbm, out_hbm,
                  s0_v, s1_v, w0_v, w1_v, wtmp_v, acc_v, ybuf_v, sem):
    wid = lax.axis_index("s") * NC + lax.axis_index("c")
    base = wid * TPW
    pltpu.sync_copy(slot_hbm.at[pl.ds(base, TPW)], s0_v)
    pltpu.sync_copy(slot_hbm.at[pl.ds(N + base, TPW)], s1_v)
    pltpu.sync_copy(tw_hbm.at[pl.ds(base, TPW)], w0_v)
    pltpu.sync_copy(tw_hbm.at[pl.ds(N + base, TPW)], w1_v)
    pltpu.sync_copy(shared_hbm.at[pl.ds(base, TPW)], acc_v)

    def accum(w_ref):
        def body(i, carry):
            ws = plsc.load_gather(w_ref, [jnp.full((L,), i, jnp.int32)])
            for j in range(D // L):
                sl = pl.ds(j * L, L)
                acc_v[i, sl] = acc_v[i, sl] + ws * ybuf_v[i, sl]
            return carry
        lax.fori_loop(0, TPW, body, 0)

    pltpu.async_copy(y_hbm.at[s0_v], ybuf_v, sem).wait()
    accum(w0_v)
    pltpu.async_copy(y_hbm.at[s1_v], ybuf_v, sem).wait()
    accum(w1_v)
    pltpu.sync_copy(acc_v, out_hbm.at[pl.ds(base, TPW)])


_sc_params = pltpu.CompilerParams(needs_layout_passes=False)

_build = pl.kernel(
    _build_body,
    out_type=[
        jax.ShapeDtypeStruct((P,), jnp.int32),
        jax.ShapeDtypeStruct((NW,), jnp.int32),
    ],
    mesh=_sc_mesh,
    scratch_types=[
        pltpu.VMEM((P,), jnp.int32),
        pltpu.VMEM((PPW,), jnp.int32),
        pltpu.VMEM((NW,), jnp.int32),
    ],
    compiler_params=_sc_params,
)

_dispatch = pl.kernel(
    _dispatch_body,
    out_type=jax.ShapeDtypeStruct((P_PAD, D), jnp.float32),
    mesh=_sc_mesh,
    scratch_types=[
        pltpu.VMEM((TPW,), jnp.int32),
        pltpu.VMEM((TPW,), jnp.int32),
        pltpu.VMEM((TPW, D), jnp.float32),
        pltpu.SemaphoreType.DMA,
    ],
    compiler_params=_sc_params,
)

_combine = pl.kernel(
    _combine_body,
    out_type=jax.ShapeDtypeStruct((N, D), jnp.float32),
    mesh=_sc_mesh,
    scratch_types=[
        pltpu.VMEM((TPW,), jnp.int32),
        pltpu.VMEM((TPW,), jnp.int32),
        pltpu.VMEM((TPW,), jnp.float32),
        pltpu.VMEM((TPW,), jnp.float32),
        pltpu.VMEM((L,), jnp.float32),
        pltpu.VMEM((TPW, D), jnp.float32),
        pltpu.VMEM((TPW, D), jnp.float32),
        pltpu.SemaphoreType.DMA,
    ],
    compiler_params=_sc_params,
)


@jax.jit
def kernel(x, Wr, br, W1s, b1s, W2s, b2s, W1e, b1e, W2e, b2e):
    flat = x.reshape(N, D)
    nt = N // TILE

    shared, w1, w2, i1, i2 = pl.pallas_call(
        _router_shared_body,
        grid=(nt,),
        in_specs=[
            pl.BlockSpec((TILE, D), lambda t: (t, 0)),
            pl.BlockSpec((E, D), lambda t: (0, 0)),
            pl.BlockSpec((1, E), lambda t: (0, 0)),
            pl.BlockSpec((DFF, D), lambda t: (0, 0)),
            pl.BlockSpec((1, DFF), lambda t: (0, 0)),
            pl.BlockSpec((D, DFF), lambda t: (0, 0)),
            pl.BlockSpec((1, D), lambda t: (0, 0)),
        ],
        out_specs=[
            pl.BlockSpec((TILE, D), lambda t: (t, 0)),
            pl.BlockSpec((TILE, 1), lambda t: (t, 0)),
            pl.BlockSpec((TILE, 1), lambda t: (t, 0)),
            pl.BlockSpec((TILE, 1), lambda t: (t, 0)),
            pl.BlockSpec((TILE, 1), lambda t: (t, 0)),
        ],
        out_shape=[
            jax.ShapeDtypeStruct((N, D), jnp.float32),
            jax.ShapeDtypeStruct((N, 1), jnp.float32),
            jax.ShapeDtypeStruct((N, 1), jnp.float32),
            jax.ShapeDtypeStruct((N, 1), jnp.int32),
            jax.ShapeDtypeStruct((N, 1), jnp.int32),
        ],
        compiler_params=pltpu.CompilerParams(
            dimension_semantics=("parallel",),
        ),
    )(flat, Wr, br.reshape(1, E), W1s, b1s.reshape(1, DFF),
      W2s, b2s.reshape(1, D))

    ids = jnp.concatenate([i1.reshape(N), i2.reshape(N)])
    tw = jnp.concatenate([w1.reshape(N), w2.reshape(N)])

    slot, te = _build(ids)
    xs = _dispatch(flat, slot)

    y = pl.pallas_call(
        _experts_body,
        grid_spec=pltpu.PrefetchScalarGridSpec(
            num_scalar_prefetch=1,
            grid=(MAX_TILES,),
            in_specs=[
                pl.BlockSpec((TILE, D), lambda t, te_r: (t, 0)),
                pl.BlockSpec((1, D, DFF), lambda t, te_r: (te_r[t], 0, 0)),
                pl.BlockSpec((1, 1, DFF), lambda t, te_r: (te_r[t], 0, 0)),
                pl.BlockSpec((1, DFF, D), lambda t, te_r: (te_r[t], 0, 0)),
                pl.BlockSpec((1, 1, D), lambda t, te_r: (te_r[t], 0, 0)),
            ],
            out_specs=pl.BlockSpec((TILE, D), lambda t, te_r: (t, 0)),
        ),
        out_shape=jax.ShapeDtypeStruct((P_PAD, D), jnp.float32),
    )(te, xs, W1e, b1e.reshape(E, 1, DFF), W2e, b2e.reshape(E, 1, D))

    out = _combine(shared, y, tw, slot)
    return out.reshape(B, S, D)


# speedup vs baseline: 1.9885x; 1.0166x over previous
import functools

import jax
import jax.numpy as jnp
from jax import lax
from jax.experimental import pallas as pl
from jax.experimental.pallas import tpu as pltpu
from jax.experimental.pallas import tpu_sc as plsc

B, S, D, DFF, E, K = 1, 2048, 768, 3072, 8, 2
N = B * S
P = K * N
TILE = 256
MAX_TILES = 24
P_PAD = MAX_TILES * TILE
NC, NS, L = 2, 16, 16
NW = NC * NS
TPW = N // NW
PPW = P // NW
_sc_mesh = plsc.VectorSubcoreMesh(
    core_axis_name="c", subcore_axis_name="s", num_cores=NC, num_subcores=NS)


def _router_shared_body(x_ref, Wr_ref, br_ref, W1s_ref, b1s_ref, W2s_ref,
                        b2s_ref, shared_ref, w1_ref, w2_ref, i1_ref, i2_ref):
    x = x_ref[...]
    logits = lax.dot_general(
        x, Wr_ref[...], (((1,), (1,)), ((), ())),
        preferred_element_type=jnp.float32) + br_ref[...]
    idx = lax.broadcasted_iota(jnp.int32, logits.shape, 1)
    m1 = jnp.max(logits, axis=-1, keepdims=True)
    i1 = jnp.min(jnp.where(logits == m1, idx, E), axis=-1, keepdims=True)
    masked = jnp.where(idx == i1, -jnp.inf, logits)
    m2 = jnp.max(masked, axis=-1, keepdims=True)
    i2 = jnp.min(jnp.where(masked == m2, idx, E), axis=-1, keepdims=True)
    b = jnp.exp(m2 - m1)
    w1 = 1.0 / (1.0 + b)
    w1_ref[...] = w1
    w2_ref[...] = 1.0 - w1
    i1_ref[...] = i1
    i2_ref[...] = i2
    xb = x.astype(jnp.bfloat16)
    h = jax.nn.gelu(lax.dot_general(
        xb, W1s_ref[...].astype(jnp.bfloat16), (((1,), (1,)), ((), ())),
        preferred_element_type=jnp.float32) + b1s_ref[...])
    shared_ref[...] = lax.dot_general(
        h.astype(jnp.bfloat16), W2s_ref[...].astype(jnp.bfloat16),
        (((1,), (1,)), ((), ())),
        preferred_element_type=jnp.float32) + b2s_ref[...]


def _lane_splat(tmp_ref, vec, lane):
    tmp_ref[...] = vec
    return plsc.load_gather(tmp_ref, [jnp.full((L,), lane, jnp.int32)])


def _build_body(ids_hbm, slot_hbm, te_hbm, ids_v, slot_v, te_v):
    cid = lax.axis_index("c")
    w = lax.axis_index("s") * NC + cid
    lane = lax.iota(jnp.int32, L)
    pltpu.sync_copy(ids_hbm, ids_v)

    def count_upto(hi):
        def body(j, tv):
            v = ids_v[pl.ds(j * L, L)]
            for e in range(E):
                s = jnp.sum(jnp.where(v == e, 1, 0).astype(jnp.int32))
                tv = tv + jnp.where(lane == e, s, 0)
            return tv
        return lax.fori_loop(0, hi, body, jnp.zeros((L,), jnp.int32))

    tot_v = count_upto(P // L)
    pref_v = count_upto(w * (PPW // L))

    base = []
    off = []
    acc = jnp.int32(0)
    for e in range(E):
        cnt = jnp.sum(jnp.where(lane == e, tot_v, 0))
        prefw = jnp.sum(jnp.where(lane == e, pref_v, 0))
        off.append(acc)
        base.append(acc + prefw)
        acc = acc + ((cnt + (TILE - 1)) // TILE) * TILE

    for i in range(PPW // L):
        v = ids_v[pl.ds(w * PPW + i * L, L)]
        slots = jnp.zeros((L,), jnp.int32)
        for e in range(E):
            mask = v == e
            mi = jnp.where(mask, 1, 0).astype(jnp.int32)
            ranks = plsc.cumsum(mi)
            slots = jnp.where(mask, base[e] + ranks - 1, slots)
            base[e] = base[e] + jnp.sum(mi)
        slot_v[pl.ds(i * L, L)] = slots
    pltpu.sync_copy(slot_v, slot_hbm.at[pl.ds(w * PPW, PPW)])

    @pl.when(w == 0)
    def _tiles():
        t0 = lane * TILE
        t1 = (lane + NS) * TILE
        te0 = jnp.full((L,), -1, jnp.int32)
        te1 = jnp.full((L,), -1, jnp.int32)
        for e in range(E):
            te0 = te0 + jnp.where(t0 >= off[e], 1, 0)
            te1 = te1 + jnp.where(t1 >= off[e], 1, 0)
        te_v[pl.ds(0, L)] = te0
        te_v[pl.ds(L, L)] = te1
        te_v[pl.ds(2 * L, L)] = jnp.where(t0 < acc, 1, 0)
        te_v[pl.ds(3 * L, L)] = jnp.where(t1 < acc, 1, 0)
        pltpu.sync_copy(te_v, te_hbm)


def _dispatch_body(x_hbm, slot_hbm, xs_hbm, idx0_v, idx1_v, xbuf, sem):
    wid = lax.axis_index("s") * NC + lax.axis_index("c")
    base = wid * TPW
    pltpu.sync_copy(slot_hbm.at[pl.ds(base, TPW)], idx0_v)
    pltpu.sync_copy(slot_hbm.at[pl.ds(N + base, TPW)], idx1_v)
    pltpu.sync_copy(x_hbm.at[pl.ds(base, TPW)], xbuf)
    pltpu.async_copy(xbuf, xs_hbm.at[idx0_v], sem).wait()
    pltpu.async_copy(xbuf, xs_hbm.at[idx1_v], sem).wait()


def _experts_body(te_ref, act_ref, xs_ref, W1e_ref, b1e_ref, W2e_ref,
                  b2e_ref, y_ref, W1b_ref, W2b_ref):
    t = pl.program_id(0)
    active = act_ref[t] == 1
    e_prev = te_ref[jnp.maximum(t - 1, 0)]
    fresh = jnp.logical_or(t == 0, te_ref[t] != e_prev)

    @pl.when(jnp.logical_and(active, fresh))
    def _cast():
        W1b_ref[...] = W1e_ref[0].astype(jnp.bfloat16)
        W2b_ref[...] = W2e_ref[0].astype(jnp.bfloat16)

    @pl.when(active)
    def _ffn():
        xb = xs_ref[...].astype(jnp.bfloat16)
        h = jax.nn.gelu(lax.dot_general(
            xb, W1b_ref[...], (((1,), (0,)), ((), ())),
            preferred_element_type=jnp.float32) + b1e_ref[0])
        y_ref[...] = lax.dot_general(
            h.astype(jnp.bfloat16), W2b_ref[...], (((1,), (0,)), ((), ())),
            preferred_element_type=jnp.float32) + b2e_ref[0]


def _combine_body(shared_hbm, y_hbm, tw_hbm, slot_hbm, out_hbm,
                  s0_v, s1_v, w0_v, w1_v, wtmp_v, acc_v, ybuf_v, sem):
    wid = lax.axis_index("s") * NC + lax.axis_index("c")
    base = wid * TPW
    pltpu.sync_copy(slot_hbm.at[pl.ds(base, TPW)], s0_v)
    pltpu.sync_copy(slot_hbm.at[pl.ds(N + base, TPW)], s1_v)
    pltpu.sync_copy(tw_hbm.at[pl.ds(base, TPW)], w0_v)
    pltpu.sync_copy(tw_hbm.at[pl.ds(N + base, TPW)], w1_v)
    pltpu.sync_copy(shared_hbm.at[pl.ds(base, TPW)], acc_v)

    def accum(w_ref):
        def body(i, carry):
            ws = plsc.load_gather(w_ref, [jnp.full((L,), i, jnp.int32)])
            for j in range(D // L):
                sl = pl.ds(j * L, L)
                acc_v[i, sl] = acc_v[i, sl] + ws * ybuf_v[i, sl]
            return carry
        lax.fori_loop(0, TPW, body, 0)

    pltpu.async_copy(y_hbm.at[s0_v], ybuf_v, sem).wait()
    accum(w0_v)
    pltpu.async_copy(y_hbm.at[s1_v], ybuf_v, sem).wait()
    accum(w1_v)
    pltpu.sync_copy(acc_v, out_hbm.at[pl.ds(base, TPW)])


_sc_params = pltpu.CompilerParams(needs_layout_passes=False)

_build = pl.kernel(
    _build_body,
    out_type=[
        jax.ShapeDtypeStruct((P,), jnp.int32),
        jax.ShapeDtypeStruct((2 * NW,), jnp.int32),
    ],
    mesh=_sc_mesh,
    scratch_types=[
        pltpu.VMEM((P,), jnp.int32),
        pltpu.VMEM((PPW,), jnp.int32),
        pltpu.VMEM((2 * NW,), jnp.int32),
    ],
    compiler_params=_sc_params,
)

_dispatch = pl.kernel(
    _dispatch_body,
    out_type=jax.ShapeDtypeStruct((P_PAD, D), jnp.float32),
    mesh=_sc_mesh,
    scratch_types=[
        pltpu.VMEM((TPW,), jnp.int32),
        pltpu.VMEM((TPW,), jnp.int32),
        pltpu.VMEM((TPW, D), jnp.float32),
        pltpu.SemaphoreType.DMA,
    ],
    compiler_params=_sc_params,
)

_combine = pl.kernel(
    _combine_body,
    out_type=jax.ShapeDtypeStruct((N, D), jnp.float32),
    mesh=_sc_mesh,
    scratch_types=[
        pltpu.VMEM((TPW,), jnp.int32),
        pltpu.VMEM((TPW,), jnp.int32),
        pltpu.VMEM((TPW,), jnp.float32),
        pltpu.VMEM((TPW,), jnp.float32),
        pltpu.VMEM((L,), jnp.float32),
        pltpu.VMEM((TPW, D), jnp.float32),
        pltpu.VMEM((TPW, D), jnp.float32),
        pltpu.SemaphoreType.DMA,
    ],
    compiler_params=_sc_params,
)


@jax.jit
def kernel(x, Wr, br, W1s, b1s, W2s, b2s, W1e, b1e, W2e, b2e):
    flat = x.reshape(N, D)
    TILE_R = 512
    nt = N // TILE_R

    shared, w1, w2, i1, i2 = pl.pallas_call(
        _router_shared_body,
        grid=(nt,),
        in_specs=[
            pl.BlockSpec((TILE_R, D), lambda t: (t, 0)),
            pl.BlockSpec((E, D), lambda t: (0, 0)),
            pl.BlockSpec((1, E), lambda t: (0, 0)),
            pl.BlockSpec((DFF, D), lambda t: (0, 0)),
            pl.BlockSpec((1, DFF), lambda t: (0, 0)),
            pl.BlockSpec((D, DFF), lambda t: (0, 0)),
            pl.BlockSpec((1, D), lambda t: (0, 0)),
        ],
        out_specs=[
            pl.BlockSpec((TILE_R, D), lambda t: (t, 0)),
            pl.BlockSpec((TILE_R, 1), lambda t: (t, 0)),
            pl.BlockSpec((TILE_R, 1), lambda t: (t, 0)),
            pl.BlockSpec((TILE_R, 1), lambda t: (t, 0)),
            pl.BlockSpec((TILE_R, 1), lambda t: (t, 0)),
        ],
        out_shape=[
            jax.ShapeDtypeStruct((N, D), jnp.float32),
            jax.ShapeDtypeStruct((N, 1), jnp.float32),
            jax.ShapeDtypeStruct((N, 1), jnp.float32),
            jax.ShapeDtypeStruct((N, 1), jnp.int32),
            jax.ShapeDtypeStruct((N, 1), jnp.int32),
        ],
        compiler_params=pltpu.CompilerParams(
            dimension_semantics=("parallel",),
        ),
    )(flat, Wr, br.reshape(1, E), W1s, b1s.reshape(1, DFF),
      W2s, b2s.reshape(1, D))

    ids = jnp.concatenate([i1.reshape(N), i2.reshape(N)])
    tw = jnp.concatenate([w1.reshape(N), w2.reshape(N)])

    slot, teact = _build(ids)
    te = teact[:NW]
    act = teact[NW:]
    xs = _dispatch(flat, slot)

    y = pl.pallas_call(
        _experts_body,
        grid_spec=pltpu.PrefetchScalarGridSpec(
            num_scalar_prefetch=2,
            grid=(MAX_TILES,),
            in_specs=[
                pl.BlockSpec((TILE, D), lambda t, te_r, a_r: (t, 0)),
                pl.BlockSpec((1, D, DFF),
                             lambda t, te_r, a_r: (te_r[t], 0, 0)),
                pl.BlockSpec((1, 1, DFF),
                             lambda t, te_r, a_r: (te_r[t], 0, 0)),
                pl.BlockSpec((1, DFF, D),
                             lambda t, te_r, a_r: (te_r[t], 0, 0)),
                pl.BlockSpec((1, 1, D),
                             lambda t, te_r, a_r: (te_r[t], 0, 0)),
            ],
            out_specs=pl.BlockSpec((TILE, D), lambda t, te_r, a_r: (t, 0)),
            scratch_shapes=[
                pltpu.VMEM((D, DFF), jnp.bfloat16),
                pltpu.VMEM((DFF, D), jnp.bfloat16),
            ],
        ),
        out_shape=jax.ShapeDtypeStruct((P_PAD, D), jnp.float32),
    )(te, act, xs, W1e, b1e.reshape(E, 1, DFF), W2e, b2e.reshape(E, 1, D))

    out = _combine(shared, y, tw, slot)
    return out.reshape(B, S, D)
